# Initial kernel scaffold; baseline (speedup 1.0000x reference)
import jax
import jax.numpy as jnp
from jax.experimental import pallas as pl


def kernel(edge_attr, W0, W1):
    raise NotImplementedError("write your pallas kernel here")


# speedup vs baseline: 4.1133x; 4.1133x over previous
import functools

import jax
import jax.numpy as jnp
from jax import lax
from jax.experimental import pallas as pl
from jax.experimental.pallas import tpu as pltpu
from jax.experimental.pallas import tpu_sc as plsc

A_ROWS = 100000
EMB = 16
N_EDGES = 3200000
NC, NS = 2, 16
NW = NC * NS
EPW = N_EDGES // NW
CHUNK = 1000
NCHUNKS = EPW // CHUNK


def _body(idx_hbm, w_hbm, out_hbm, idx_v, rows_v, out_v, sem):
    wid = lax.axis_index("s") * NC + lax.axis_index("c")

    def chunk_body(j, carry):
        base_e = wid * EPW + j * CHUNK
        pltpu.sync_copy(idx_hbm.at[pl.ds(2 * base_e, 2 * CHUNK)], idx_v)
        pltpu.async_copy(w_hbm.at[idx_v], rows_v, sem).wait()

        def add_body(i, c):
            out_v[i] = rows_v[2 * i] + rows_v[2 * i + 1]
            return c

        lax.fori_loop(0, CHUNK, add_body, 0, unroll=4)
        pltpu.sync_copy(out_v, out_hbm.at[pl.ds(base_e, CHUNK)])
        return carry

    lax.fori_loop(0, NCHUNKS, chunk_body, 0)


_gather_sum = functools.partial(
    pl.kernel,
    mesh=plsc.VectorSubcoreMesh(core_axis_name="c", subcore_axis_name="s"),
    out_type=jax.ShapeDtypeStruct((N_EDGES, EMB), jnp.float32),
    scratch_types=[
        pltpu.VMEM((2 * CHUNK,), jnp.int32),
        pltpu.VMEM((2 * CHUNK, EMB), jnp.float32),
        pltpu.VMEM((CHUNK, EMB), jnp.float32),
        pltpu.SemaphoreType.DMA,
    ],
    compiler_params=pltpu.CompilerParams(use_tc_tiling_on_sc=False),
)(_body)


def kernel(edge_attr, W0, W1):
    if edge_attr.ndim == 1:
        edge_attr = edge_attr[:, None]
    idx = (edge_attr.astype(jnp.int32)
           + jnp.array([0, A_ROWS], jnp.int32)[None, :]).reshape(-1)
    w = jnp.concatenate([W0, W1], axis=0)
    return _gather_sum(idx, w)


# speedup vs baseline: 15.9096x; 3.8678x over previous
import functools

import jax
import jax.numpy as jnp
from jax import lax
from jax.experimental import pallas as pl
from jax.experimental.pallas import tpu as pltpu
from jax.experimental.pallas import tpu_sc as plsc

A_ROWS = 100000
EMB = 16
N_EDGES = 3200000
NC, NS = 2, 16
NW = NC * NS

CH_EDGES = 1024
NCHUNKS = N_EDGES // CH_EDGES
K_ITERS = -(-NCHUNKS // NW)
HALF = N_EDGES * 8


def _body(x_hbm, w_hbm, out_hbm, idx_v, rows_v, outt_v, sem):
    wid = lax.axis_index("s") * NC + lax.axis_index("c")
    iota = lax.iota(jnp.int32, 16)
    voff = (iota >> 3) * 8192 + (iota & 7) * 128

    def chunk_body(k, carry):
        c = wid + NW * k

        @pl.when(c < NCHUNKS)
        def _():
            pltpu.sync_copy(x_hbm.at[pl.ds(c * 2048, 2048)], idx_v)

            def off_body(i, cc):
                off = (i >> 3) * 256 + 128 + ((i & 7) << 4)
                idx_v[pl.ds(off, 16)] = idx_v[pl.ds(off, 16)] + A_ROWS
                return cc

            lax.fori_loop(0, 64, off_body, 0, unroll=8)

            pltpu.async_copy(w_hbm.at[idx_v], rows_v, sem).wait()

            def add_body(j, cc):
                t = j >> 7
                i0 = (t << 7) + j
                val = rows_v[i0] + rows_v[i0 + 128]
                pos = voff + (j + t * 896)
                plsc.store_scatter(outt_v, [pos], val)
                return cc

            lax.fori_loop(0, CH_EDGES, add_body, 0, unroll=4)

            pltpu.sync_copy(outt_v.at[pl.ds(0, 8192)],
                            out_hbm.at[pl.ds(c * 8192, 8192)])
            pltpu.sync_copy(outt_v.at[pl.ds(8192, 8192)],
                            out_hbm.at[pl.ds(HALF + c * 8192, 8192)])

        return carry

    lax.fori_loop(0, K_ITERS, chunk_body, 0)


_gather_sum = functools.partial(
    pl.kernel,
    mesh=plsc.VectorSubcoreMesh(core_axis_name="c", subcore_axis_name="s"),
    out_type=jax.ShapeDtypeStruct((N_EDGES * EMB,), jnp.float32),
    scratch_types=[
        pltpu.VMEM((2 * CH_EDGES,), jnp.int32),
        pltpu.VMEM((2 * CH_EDGES, EMB), jnp.float32),
        pltpu.VMEM((CH_EDGES * EMB,), jnp.float32),
        pltpu.SemaphoreType.DMA,
    ],
    compiler_params=pltpu.CompilerParams(use_tc_tiling_on_sc=False,
                                         needs_layout_passes=False),
)(_body)


def kernel(edge_attr, W0, W1):
    if edge_attr.ndim == 1:
        edge_attr = edge_attr[:, None]
    x1d = (edge_attr.astype(jnp.int32)
           .reshape(N_EDGES // 128, 128, 2)
           .transpose(0, 2, 1)
           .reshape(-1))
    w = jnp.concatenate([W0, W1], axis=0)
    outf = _gather_sum(x1d, w)
    return (outf.reshape(2, N_EDGES // 128, 8, 128)
            .transpose(1, 3, 0, 2)
            .reshape(N_EDGES, EMB))


# speedup vs baseline: 20.2152x; 1.2706x over previous
import functools

import jax
import jax.numpy as jnp
from jax import lax
from jax.experimental import pallas as pl
from jax.experimental.pallas import tpu as pltpu
from jax.experimental.pallas import tpu_sc as plsc

A_ROWS = 100000
EMB = 16
N_EDGES = 3200000
NC, NS = 2, 16
NW = NC * NS

CH_EDGES = 1024
NTILES = CH_EDGES // 128
NCHUNKS = N_EDGES // CH_EDGES
K_ITERS = -(-NCHUNKS // NW)
HALF = N_EDGES * 8


def _body(x_hbm, w_hbm, out_hbm, idx_v, rows_v, outt_v, sem_i, sem_g, sem_o):
    wid = lax.axis_index("s") * NC + lax.axis_index("c")
    iota = lax.iota(jnp.int32, 16)
    voff = (iota >> 3) * (NTILES * 1024) + (iota & 7) * 128
    vofft = [voff + t * 1024 for t in range(NTILES)]

    def chunk_of(k):
        return wid + NW * k

    def idx_start(k, b):
        pltpu.async_copy(x_hbm.at[pl.ds(chunk_of(k) * 2048, 2048)],
                         idx_v.at[b], sem_i[b])

    def idx_wait(b):
        pltpu.make_async_copy(x_hbm.at[pl.ds(0, 2048)],
                              idx_v.at[b], sem_i[b]).wait()

    def offset_pass(b):
        def off_body(i, cc):
            off = (i >> 3) * 256 + 128 + ((i & 7) << 4)
            idx_v[b, pl.ds(off, 16)] = idx_v[b, pl.ds(off, 16)] + A_ROWS
            return cc
        lax.fori_loop(0, 64, off_body, 0, unroll=8)

    def gather_start(b):
        pltpu.async_copy(w_hbm.at[idx_v.at[b]], rows_v.at[b], sem_g[b])

    def gather_wait(b):
        pltpu.make_async_copy(w_hbm.at[idx_v.at[b]],
                              rows_v.at[b], sem_g[b]).wait()

    def add_scatter(b):
        for t in range(NTILES):
            base = t * 256

            def inner(jl, posv):
                val = rows_v[b, base + jl] + rows_v[b, base + 128 + jl]
                plsc.store_scatter(outt_v.at[b], [posv], val)
                return posv + 1

            lax.fori_loop(0, 128, inner, vofft[t], unroll=8)

    def out_start(k, b):
        c = chunk_of(k)
        pltpu.async_copy(outt_v.at[b, pl.ds(0, 8192)],
                         out_hbm.at[pl.ds(c * 8192, 8192)], sem_o[b])
        pltpu.async_copy(outt_v.at[b, pl.ds(8192, 8192)],
                         out_hbm.at[pl.ds(HALF + c * 8192, 8192)], sem_o[b])

    def out_wait(b):
        for _ in range(2):
            pltpu.make_async_copy(outt_v.at[b, pl.ds(0, 8192)],
                                  out_hbm.at[pl.ds(0, 8192)],
                                  sem_o[b]).wait()

    @pl.when(chunk_of(0) < NCHUNKS)
    def _():
        idx_start(0, 0)

    def block(k, b, carry):
        valid_k = chunk_of(k) < NCHUNKS

        @pl.when(valid_k)
        def _():
            idx_wait(b)
            offset_pass(b)
            gather_start(b)

        @pl.when((k >= 1) & (chunk_of(k - 1) < NCHUNKS))
        def _():
            gather_wait(1 - b)
            @pl.when(k >= 3)
            def _():
                out_wait(1 - b)
            add_scatter(1 - b)
            out_start(k - 1, 1 - b)

        @pl.when(chunk_of(k + 1) < NCHUNKS)
        def _():
            idx_start(k + 1, 1 - b)

        return carry

    def block_pair(m, carry):
        k = 2 * m
        block(k, 0, carry)
        block(k + 1, 1, carry)
        return carry

    lax.fori_loop(0, K_ITERS // 2 + 1, block_pair, 0)

    @pl.when(chunk_of(K_ITERS - 2) < NCHUNKS)
    def _():
        out_wait((K_ITERS - 2) % 2)

    @pl.when(chunk_of(K_ITERS - 1) < NCHUNKS)
    def _():
        out_wait((K_ITERS - 1) % 2)


_gather_sum = functools.partial(
    pl.kernel,
    mesh=plsc.VectorSubcoreMesh(core_axis_name="c", subcore_axis_name="s"),
    out_type=jax.ShapeDtypeStruct((N_EDGES * EMB,), jnp.float32),
    scratch_types=[
        pltpu.VMEM((2, 2 * CH_EDGES), jnp.int32),
        pltpu.VMEM((2, 2 * CH_EDGES, EMB), jnp.float32),
        pltpu.VMEM((2, CH_EDGES * EMB), jnp.float32),
        [pltpu.SemaphoreType.DMA] * 2,
        [pltpu.SemaphoreType.DMA] * 2,
        [pltpu.SemaphoreType.DMA] * 2,
    ],
    compiler_params=pltpu.CompilerParams(use_tc_tiling_on_sc=False,
                                         needs_layout_passes=False),
)(_body)


def kernel(edge_attr, W0, W1):
    if edge_attr.ndim == 1:
        edge_attr = edge_attr[:, None]
    x1d = (edge_attr.astype(jnp.int32)
           .reshape(N_EDGES // 128, 128, 2)
           .transpose(0, 2, 1)
           .reshape(-1))
    w = jnp.concatenate([W0, W1], axis=0)
    outf = _gather_sum(x1d, w)
    return (outf.reshape(2, N_EDGES // 128, 8, 128)
            .transpose(1, 3, 0, 2)
            .reshape(N_EDGES, EMB))
